# Initial kernel scaffold; baseline (speedup 1.0000x reference)
import jax
import jax.numpy as jnp
from jax.experimental import pallas as pl


def kernel(x, table, W_int, b_int, W_lin):
    raise NotImplementedError("write your pallas kernel here")


# speedup vs baseline: 6.1749x; 6.1749x over previous
import functools

import jax
import jax.numpy as jnp
import numpy as np
from jax import lax
from jax.experimental import pallas as pl
from jax.experimental.pallas import tpu as pltpu
from jax.experimental.pallas import tpu_sc as plsc

_NUM_FIELDS = 26
_EMBED_DIM = 64
_BATCH = 4096
_FIELD_DIMS = [1000] * _NUM_FIELDS
_OFFSETS = np.concatenate(([0], np.cumsum(_FIELD_DIMS)[:-1])).astype(np.int32)
_ROW = []
_COL = []
for _i in range(_NUM_FIELDS - 1):
    for _j in range(_i + 1, _NUM_FIELDS):
        _ROW.append(_i)
        _COL.append(_j)

_NC, _NS = 2, 16
_NW = _NC * _NS
_ROWS_TOTAL = _BATCH * _NUM_FIELDS
_ROWS_PER_W = _ROWS_TOTAL // _NW
_CHUNK = 128
_NCHUNK = _ROWS_PER_W // _CHUNK


def _sc_gather(table, idx3):
    mesh = plsc.VectorSubcoreMesh(core_axis_name="c", subcore_axis_name="s")

    @functools.partial(
        pl.kernel,
        out_type=jax.ShapeDtypeStruct((_ROWS_TOTAL, _EMBED_DIM), jnp.float32),
        mesh=mesh,
        scratch_types=[
            pltpu.VMEM((_NCHUNK, _CHUNK), jnp.int32),
            pltpu.VMEM((2, _CHUNK, _EMBED_DIM), jnp.float32),
            pltpu.SemaphoreType.DMA,
            pltpu.SemaphoreType.DMA,
        ],
        compiler_params=pltpu.CompilerParams(use_tc_tiling_on_sc=False),
    )
    def k(table_hbm, idx_hbm, out_hbm, idx_v, rows_v, sem0, sem1):
        wid = lax.axis_index("s") * _NC + lax.axis_index("c")
        base = wid * _ROWS_PER_W
        pltpu.sync_copy(idx_hbm.at[wid], idx_v)
        sems = (sem0, sem1)
        cps = [None, None]
        for j in range(_NCHUNK):
            b = j % 2
            cps[b] = pltpu.async_copy(
                table_hbm.at[idx_v.at[j]], rows_v.at[b], sems[b])
            if j >= 1:
                pb = (j - 1) % 2
                cps[pb].wait()
                pltpu.sync_copy(
                    rows_v.at[pb],
                    out_hbm.at[pl.ds(base + (j - 1) * _CHUNK, _CHUNK)])
        cps[(_NCHUNK - 1) % 2].wait()
        pltpu.sync_copy(
            rows_v.at[(_NCHUNK - 1) % 2],
            out_hbm.at[pl.ds(base + (_NCHUNK - 1) * _CHUNK, _CHUNK)])

    return k(table, idx3)


_BB = 512


def _tc_body(e_ref, k_ref, wl_ref, b_ref, o_ref):
    e = e_ref[...]
    g = jnp.dot(e.astype(jnp.bfloat16), k_ref[...],
                preferred_element_type=jnp.float32)
    t = e * (g + wl_ref[...])
    logit = jnp.sum(t, axis=1) + b_ref[0]
    o_ref[...] = jax.nn.sigmoid(logit)


def _tc_fwfm(embed, kb, w_lin, b_int):
    d = _NUM_FIELDS * _EMBED_DIM
    grid = (_BATCH // _BB,)
    return pl.pallas_call(
        _tc_body,
        grid=grid,
        in_specs=[
            pl.BlockSpec((_BB, d), lambda g: (g, 0)),
            pl.BlockSpec((d, d), lambda g: (0, 0)),
            pl.BlockSpec((1, d), lambda g: (0, 0)),
            pl.BlockSpec(memory_space=pltpu.SMEM),
        ],
        out_specs=pl.BlockSpec((_BB,), lambda g: (g,)),
        out_shape=jax.ShapeDtypeStruct((_BATCH,), jnp.float32),
    )(embed, kb, w_lin, b_int)


def kernel(x, table, W_int, b_int, W_lin):
    idx = (x + jnp.asarray(_OFFSETS)[None, :]).reshape(-1)
    idx3 = idx.reshape(_NW, _NCHUNK, _CHUNK)
    embed = _sc_gather(table, idx3)
    embed = embed.reshape(_BATCH, _NUM_FIELDS * _EMBED_DIM)

    row = jnp.asarray(_ROW, dtype=jnp.int32)
    col = jnp.asarray(_COL, dtype=jnp.int32)
    m = jnp.zeros((_NUM_FIELDS, _NUM_FIELDS), jnp.float32)
    m = m.at[col, row].set(W_int[0, :])
    kb = jnp.kron(m, jnp.eye(_EMBED_DIM, dtype=jnp.float32))
    kb = kb.astype(jnp.bfloat16)

    return _tc_fwfm(embed, kb, W_lin, b_int)
